# Initial kernel scaffold; baseline (speedup 1.0000x reference)
import jax
import jax.numpy as jnp
from jax.experimental import pallas as pl


def kernel(projections, memory):
    raise NotImplementedError("write your pallas kernel here")


# speedup vs baseline: 5.3023x; 5.3023x over previous
import functools

import jax
import jax.numpy as jnp
from jax.experimental import pallas as pl
from jax.experimental.pallas import tpu as pltpu

_B, _S, _D = 4, 8192, 768
_K = 128
_N = _B * _S

_TN1 = 1024
_TN2 = 8192
_TN4 = 1024


def _pass1_kernel(mem_ref, proj_ref, a_ref, s1_ref, mn_ref, acc_ref):
    i = pl.program_id(0)
    nb = pl.num_programs(0)

    @pl.when(i == 0)
    def _():
        m = mem_ref[...]
        sq = jnp.sum(m * m, axis=1, keepdims=True)
        mn_ref[...] = m * jax.lax.rsqrt(jnp.maximum(sq, 1e-12))
        acc_ref[...] = jnp.zeros_like(acc_ref)

    p = proj_ref[...]
    sq = jnp.sum(p * p, axis=1, keepdims=True)
    pn = p * jax.lax.rsqrt(jnp.maximum(sq, 1e-12))
    sim = jnp.dot(pn, mn_ref[...].T, preferred_element_type=jnp.float32)
    a = jnp.exp(sim / 0.05)
    a_ref[...] = a
    acc_ref[...] += jnp.sum(a, axis=0, keepdims=True)

    @pl.when(i == nb - 1)
    def _():
        s1_ref[...] = acc_ref[...]


def _sink_pass_kernel(sprev_ref, a_ref, snext_ref, u_ref, acc_ref):
    i = pl.program_id(0)
    nb = pl.num_programs(0)

    @pl.when(i == 0)
    def _():
        u_ref[...] = 1.0 / (_K * sprev_ref[...])
        acc_ref[...] = jnp.zeros_like(acc_ref)

    a = a_ref[...]
    t = jnp.sum(a * u_ref[...], axis=1, keepdims=True)
    v = 1.0 / (_N * t)
    acc_ref[...] += jnp.sum(a * v, axis=0, keepdims=True)

    @pl.when(i == nb - 1)
    def _():
        snext_ref[...] = acc_ref[...]


def _pass4_kernel(s3_ref, a_ref, proj_ref, mem_ref, out_ref, u_ref):
    i = pl.program_id(0)

    @pl.when(i == 0)
    def _():
        u_ref[...] = 1.0 / (_K * s3_ref[...])

    scores = a_ref[...] * u_ref[...]
    idx = jnp.argmax(scores, axis=1)
    iota = jax.lax.broadcasted_iota(jnp.int32, scores.shape, 1)
    onehot = (iota == idx[:, None]).astype(jnp.float32)
    assign = jnp.dot(onehot, mem_ref[...], preferred_element_type=jnp.float32)
    out_ref[...] = (proj_ref[...] + assign) * 0.5


@jax.jit
def kernel(projections, memory):
    bsz, seq, d = projections.shape
    proj = projections.reshape(-1, d)
    wmem = memory[:_K, :]

    a, s1 = pl.pallas_call(
        _pass1_kernel,
        grid=(_N // _TN1,),
        in_specs=[
            pl.BlockSpec((_K, _D), lambda i: (0, 0)),
            pl.BlockSpec((_TN1, _D), lambda i: (i, 0)),
        ],
        out_specs=[
            pl.BlockSpec((_TN1, _K), lambda i: (i, 0)),
            pl.BlockSpec((1, _K), lambda i: (0, 0)),
        ],
        out_shape=[
            jax.ShapeDtypeStruct((_N, _K), jnp.float32),
            jax.ShapeDtypeStruct((1, _K), jnp.float32),
        ],
        scratch_shapes=[
            pltpu.VMEM((_K, _D), jnp.float32),
            pltpu.VMEM((1, _K), jnp.float32),
        ],
    )(wmem, proj)

    sink = pl.pallas_call(
        _sink_pass_kernel,
        grid=(_N // _TN2,),
        in_specs=[
            pl.BlockSpec((1, _K), lambda i: (0, 0)),
            pl.BlockSpec((_TN2, _K), lambda i: (i, 0)),
        ],
        out_specs=pl.BlockSpec((1, _K), lambda i: (0, 0)),
        out_shape=jax.ShapeDtypeStruct((1, _K), jnp.float32),
        scratch_shapes=[
            pltpu.VMEM((1, _K), jnp.float32),
            pltpu.VMEM((1, _K), jnp.float32),
        ],
    )
    s2 = sink(s1, a)
    s3 = sink(s2, a)

    out = pl.pallas_call(
        _pass4_kernel,
        grid=(_N // _TN4,),
        in_specs=[
            pl.BlockSpec((1, _K), lambda i: (0, 0)),
            pl.BlockSpec((_TN4, _K), lambda i: (i, 0)),
            pl.BlockSpec((_TN4, _D), lambda i: (i, 0)),
            pl.BlockSpec((_K, _D), lambda i: (0, 0)),
        ],
        out_specs=pl.BlockSpec((_TN4, _D), lambda i: (i, 0)),
        out_shape=jax.ShapeDtypeStruct((_N, _D), jnp.float32),
        scratch_shapes=[pltpu.VMEM((1, _K), jnp.float32)],
    )(s3, a, proj, wmem)

    return out.reshape(bsz, seq, d)


# speedup vs baseline: 5.9538x; 1.1229x over previous
import jax
import jax.numpy as jnp
from jax.experimental import pallas as pl
from jax.experimental.pallas import tpu as pltpu

_B, _S, _D = 4, 8192, 768
_K = 128
_N = _B * _S

_TN = 1024
_NB = _N // _TN
_CH = 2048


def _fused_kernel(mem_ref, proj_ref, out_ref, mn_ref, a_ref, s1_ref, u3_ref):
    i = pl.program_id(0)

    @pl.when(i == 0)
    def _():
        m = mem_ref[...]
        sq = jnp.sum(m * m, axis=1, keepdims=True)
        mn_ref[...] = m * jax.lax.rsqrt(jnp.maximum(sq, 1e-12))
        s1_ref[...] = jnp.zeros_like(s1_ref)

    @pl.when(i < _NB)
    def _():
        p = proj_ref[...]
        sq = jnp.sum(p * p, axis=1, keepdims=True)
        pn = p * jax.lax.rsqrt(jnp.maximum(sq, 1e-12))
        sim = jnp.dot(pn, mn_ref[...].T, preferred_element_type=jnp.float32)
        a = jnp.exp(sim / 0.05)
        a_ref[pl.ds(i * _TN, _TN), :] = a
        s1_ref[...] += jnp.sum(a, axis=0, keepdims=True)

    @pl.when(i == _NB)
    def _():
        def sink(u):
            def body(c, acc):
                a = a_ref[pl.ds(c * _CH, _CH), :]
                t = jnp.sum(a * u, axis=1, keepdims=True)
                v = 1.0 / (_N * t)
                return acc + jnp.sum(a * v, axis=0, keepdims=True)
            s = jax.lax.fori_loop(0, _N // _CH, body,
                                  jnp.zeros((1, _K), jnp.float32))
            return 1.0 / (_K * s)

        u1 = 1.0 / (_K * s1_ref[...])
        u2 = sink(u1)
        u3_ref[...] = sink(u2)

    @pl.when(i >= _NB)
    def _():
        j = i - _NB
        a = a_ref[pl.ds(j * _TN, _TN), :]
        scores = a * u3_ref[...]
        idx = jnp.argmax(scores, axis=1)
        iota = jax.lax.broadcasted_iota(jnp.int32, scores.shape, 1)
        onehot = (iota == idx[:, None]).astype(jnp.float32)
        assign = jnp.dot(onehot, mem_ref[...], preferred_element_type=jnp.float32)
        out_ref[...] = (proj_ref[...] + assign) * 0.5


@jax.jit
def kernel(projections, memory):
    bsz, seq, d = projections.shape
    proj = projections.reshape(-1, d)
    wmem = memory[:_K, :]

    out = pl.pallas_call(
        _fused_kernel,
        grid=(2 * _NB,),
        in_specs=[
            pl.BlockSpec((_K, _D), lambda i: (0, 0)),
            pl.BlockSpec((_TN, _D), lambda i: (jnp.where(i < _NB, i, i - _NB), 0)),
        ],
        out_specs=pl.BlockSpec((_TN, _D),
                               lambda i: (jnp.where(i < _NB, 0, i - _NB), 0)),
        out_shape=jax.ShapeDtypeStruct((_N, _D), jnp.float32),
        scratch_shapes=[
            pltpu.VMEM((_K, _D), jnp.float32),
            pltpu.VMEM((_N, _K), jnp.float32),
            pltpu.VMEM((1, _K), jnp.float32),
            pltpu.VMEM((1, _K), jnp.float32),
        ],
    )(wmem, proj)

    return out.reshape(bsz, seq, d)


# speedup vs baseline: 6.7231x; 1.1292x over previous
import jax
import jax.numpy as jnp
from jax.experimental import pallas as pl
from jax.experimental.pallas import tpu as pltpu

_B, _S, _D = 4, 8192, 768
_K = 128
_N = _B * _S

_TN = 2048
_NB = _N // _TN
_CH = 2048


def _fused_kernel(mem_ref, proj_ref, out_ref, mn_ref, a_ref, s1_ref, u3_ref):
    i = pl.program_id(0)

    @pl.when(i == 0)
    def _():
        m = mem_ref[...]
        sq = jnp.sum(m * m, axis=1, keepdims=True)
        mn_ref[...] = m * jax.lax.rsqrt(jnp.maximum(sq, 1e-12))
        s1_ref[...] = jnp.zeros_like(s1_ref)

    @pl.when(i < _NB)
    def _():
        p = proj_ref[...]
        sq = jnp.sum(p * p, axis=1, keepdims=True)
        pn = p * jax.lax.rsqrt(jnp.maximum(sq, 1e-12))
        sim = jnp.dot(pn, mn_ref[...].T, preferred_element_type=jnp.float32)
        a = jnp.exp(sim / 0.05)
        a_ref[pl.ds(i * _TN, _TN), :] = a
        s1_ref[...] += jnp.sum(a, axis=0, keepdims=True)

    @pl.when(i == _NB)
    def _():
        def sink(u):
            def body(c, acc):
                a = a_ref[pl.ds(c * _CH, _CH), :]
                t = jnp.sum(a * u, axis=1, keepdims=True)
                v = 1.0 / (_N * t)
                return acc + jnp.sum(a * v, axis=0, keepdims=True)
            s = jax.lax.fori_loop(0, _N // _CH, body,
                                  jnp.zeros((1, _K), jnp.float32))
            return 1.0 / (_K * s)

        u1 = 1.0 / (_K * s1_ref[...])
        u2 = sink(u1)
        u3_ref[...] = sink(u2)

    @pl.when(i >= _NB)
    def _():
        j = i - _NB
        a = a_ref[pl.ds(j * _TN, _TN), :]
        scores = a * u3_ref[...]
        idx = jnp.argmax(scores, axis=1)
        iota = jax.lax.broadcasted_iota(jnp.int32, scores.shape, 1)
        onehot = (iota == idx[:, None]).astype(jnp.float32)
        assign = jnp.dot(onehot, mem_ref[...], preferred_element_type=jnp.float32)
        out_ref[...] = (proj_ref[...] + assign) * 0.5


@jax.jit
def kernel(projections, memory):
    bsz, seq, d = projections.shape
    proj = projections.reshape(-1, d)
    wmem = memory[:_K, :]

    out = pl.pallas_call(
        _fused_kernel,
        grid=(2 * _NB,),
        in_specs=[
            pl.BlockSpec((_K, _D), lambda i: (0, 0)),
            pl.BlockSpec((_TN, _D), lambda i: (jnp.where(i < _NB, i, i - _NB), 0)),
        ],
        out_specs=pl.BlockSpec((_TN, _D),
                               lambda i: (jnp.where(i < _NB, 0, i - _NB), 0)),
        out_shape=jax.ShapeDtypeStruct((_N, _D), jnp.float32),
        scratch_shapes=[
            pltpu.VMEM((_K, _D), jnp.float32),
            pltpu.VMEM((_N, _K), jnp.float32),
            pltpu.VMEM((1, _K), jnp.float32),
            pltpu.VMEM((1, _K), jnp.float32),
        ],
    )(wmem, proj)

    return out.reshape(bsz, seq, d)
